# baseline (speedup 1.0000x reference)
import functools

import jax
import jax.numpy as jnp
from jax import lax
from jax.experimental import pallas as pl
from jax.experimental.pallas import tpu as pltpu
from jax.experimental.pallas import tpu_sc as plsc

_BLOCK = 4096
_N = 32768
_NUM_EMB = 512
_DIM = 32
_NC = 2
_NS = 16
_NW = _NC * _NS
_B_PER_W = _N // _NW


def _vq_idx_kernel(ze_ref, cbt_ref, out_ref):
    ze = ze_ref[...]
    cbt = cbt_ref[...]
    cb_norm = jnp.sum(cbt * cbt, axis=0)[None, :]
    dist = cb_norm - 2.0 * jax.lax.dot_general(
        ze, cbt, (((1,), (0,)), ((), ())), preferred_element_type=jnp.float32
    )
    min_d = jnp.min(dist, axis=1, keepdims=True)
    iota = jax.lax.broadcasted_iota(jnp.int32, dist.shape, 1)
    idxm = jnp.where(dist == min_d, iota, dist.shape[1])
    out_ref[...] = jnp.min(idxm, axis=1, keepdims=True)


_CHUNK = 256


def _sc_gather_kernel(table_hbm, idx_hbm, out_hbm, idx_v, rows_v, sem):
    wid = lax.axis_index("s") * _NC + lax.axis_index("c")
    base = wid * _B_PER_W
    pltpu.sync_copy(idx_hbm.at[pl.ds(base, _B_PER_W)], idx_v)
    for j in range(_B_PER_W // _CHUNK):
        off = j * _CHUNK
        pltpu.async_copy(
            table_hbm.at[idx_v.at[pl.ds(off, _CHUNK)]], rows_v, sem
        ).wait()
        pltpu.sync_copy(rows_v, out_hbm.at[pl.ds(base + off, _CHUNK)])


_sc_gather = functools.partial(
    pl.kernel,
    mesh=plsc.VectorSubcoreMesh(core_axis_name="c", subcore_axis_name="s"),
    out_type=jax.ShapeDtypeStruct((_N, 128), jnp.float32),
    scratch_types=[
        pltpu.VMEM((_B_PER_W,), jnp.int32),
        pltpu.VMEM((256, 128), jnp.float32),
        pltpu.SemaphoreType.DMA,
    ],
)(_sc_gather_kernel)


@jax.jit
def kernel(x, code_book):
    b, h, w, c = x.shape
    n = b * h * w
    ze = x.reshape(n, c)
    num_emb = code_book.shape[0]
    idx = pl.pallas_call(
        _vq_idx_kernel,
        grid=(n // _BLOCK,),
        in_specs=[
            pl.BlockSpec((_BLOCK, c), lambda i: (i, 0)),
            pl.BlockSpec((c, num_emb), lambda i: (0, 0)),
        ],
        out_specs=pl.BlockSpec((_BLOCK, 1), lambda i: (i, 0)),
        out_shape=jax.ShapeDtypeStruct((n, 1), jnp.int32),
        compiler_params=pltpu.CompilerParams(
            dimension_semantics=("parallel",),
        ),
    )(ze, code_book.T)
    cb_pad = jnp.pad(code_book, ((0, 0), (0, 128 - c)))
    zq = _sc_gather(cb_pad, idx.reshape(n))
    return zq[:, :c].reshape(b, h, w, c)


# speedup vs baseline: 4.5817x; 4.5817x over previous
import jax
import jax.numpy as jnp
from jax.experimental import pallas as pl
from jax.experimental.pallas import tpu as pltpu

_BLOCK = 4096


def _vq_block_kernel(ze_ref, cbt_ref, cb_ref, out_ref):
    ze = ze_ref[...]
    cbt = cbt_ref[...]
    cb = cb_ref[...]
    cb_norm = jnp.sum(cbt * cbt, axis=0)[None, :]
    dist = cb_norm - 2.0 * jax.lax.dot_general(
        ze, cbt, (((1,), (0,)), ((), ())), preferred_element_type=jnp.float32
    )
    min_d = jnp.min(dist, axis=1, keepdims=True)
    hot = jnp.where(dist == min_d, 1.0, 0.0)
    count = jnp.sum(hot, axis=1, keepdims=True)
    zq = jax.lax.dot_general(
        hot, cb, (((1,), (0,)), ((), ())), preferred_element_type=jnp.float32
    )
    out_ref[...] = zq / count


@jax.jit
def kernel(x, code_book):
    b, h, w, c = x.shape
    n = b * h * w
    ze = x.reshape(n, c)
    num_emb = code_book.shape[0]
    zq = pl.pallas_call(
        _vq_block_kernel,
        grid=(n // _BLOCK,),
        in_specs=[
            pl.BlockSpec((_BLOCK, c), lambda i: (i, 0)),
            pl.BlockSpec((c, num_emb), lambda i: (0, 0)),
            pl.BlockSpec((num_emb, c), lambda i: (0, 0)),
        ],
        out_specs=pl.BlockSpec((_BLOCK, c), lambda i: (i, 0)),
        out_shape=jax.ShapeDtypeStruct((n, c), x.dtype),
        compiler_params=pltpu.CompilerParams(
            dimension_semantics=("parallel",),
        ),
    )(ze, code_book.T, code_book)
    return zq.reshape(b, h, w, c)
